# baseline (speedup 1.0000x reference)
import functools

import jax
import jax.numpy as jnp
from jax import lax
from jax.experimental import pallas as pl
from jax.experimental.pallas import tpu as pltpu
from jax.experimental.pallas import tpu_sc as plsc

ALPHA = 0.2
BATCH = 16384
EMBED_DIM = 64
CHUNK = 128
NGROUP = CHUNK // 16


def _sc_body(e_hbm, rpad_hbm, s_hbm, r_hbm, o_hbm, out_hbm,
             s_vm, o_vm, r_i, sval, oval, rrow, outbuf, sems):
    info = plsc.get_sparse_core_info()
    nw = info.num_cores * info.num_subcores
    b_per_w = BATCH // nw
    nchunk = b_per_w // CHUNK

    wid = lax.axis_index("s") * info.num_cores + lax.axis_index("c")
    lanes = lax.iota(jnp.int32, 16)

    pltpu.sync_copy(s_hbm.at[wid], s_vm)
    pltpu.sync_copy(o_hbm.at[wid], o_vm)
    pltpu.sync_copy(r_hbm.at[wid], r_i)

    def issue(c, buf):
        pltpu.async_copy(rpad_hbm.at[r_i.at[c]], rrow.at[buf], sems.at[buf])

        def issue_d(d, carry):
            pltpu.async_copy(e_hbm.at[d].at[s_vm.at[c]],
                             sval.at[buf].at[d], sems.at[buf])
            pltpu.async_copy(e_hbm.at[d].at[o_vm.at[c]],
                             oval.at[buf].at[d], sems.at[buf])
            return carry

        lax.fori_loop(0, EMBED_DIM, issue_d, 0)

    def drain(buf):
        pltpu.make_async_copy(e_hbm.at[:, pl.ds(0, CHUNK)], sval.at[buf],
                              sems.at[buf]).wait()
        pltpu.make_async_copy(e_hbm.at[:, pl.ds(0, CHUNK)], oval.at[buf],
                              sems.at[buf]).wait()
        pltpu.make_async_copy(rpad_hbm.at[pl.ds(0, CHUNK)], rrow.at[buf],
                              sems.at[buf]).wait()

    def compute(c, buf):
        for bg in range(NGROUP):
            bsl = pl.ds(bg * 16, 16)
            brow = bg * 16 + lanes

            def d_body(d, acc):
                dv = jnp.full((16,), 0, jnp.int32) + d
                sv = sval[buf, d, bsl]
                ov = oval[buf, d, bsl]
                rc = plsc.load_gather(rrow.at[buf], [brow, dv])
                ro = plsc.load_gather(rrow.at[buf], [brow, dv + EMBED_DIM])
                t = ov - sv - rc
                off = jnp.maximum(ro, 0.0)
                a = jnp.abs(t)
                return acc + (jnp.maximum(a - off, 0.0)
                              + ALPHA * jnp.minimum(a, off))

            acc = lax.fori_loop(0, EMBED_DIM, d_body,
                                jnp.zeros((16,), jnp.float32))
            outbuf[c, pl.ds(bg * 16, 16)] = -acc

    issue(0, 0)

    def chunk_body(c, carry):
        buf = c % 2

        @pl.when(c + 1 < nchunk)
        def _():
            issue(c + 1, 1 - buf)

        drain(buf)
        compute(c, buf)
        return carry

    lax.fori_loop(0, nchunk, chunk_body, 0)
    pltpu.sync_copy(outbuf, out_hbm.at[wid])


def kernel(E_center, R_center, R_offset, s, r, o):
    info = plsc.get_sparse_core_info()
    nw = info.num_cores * info.num_subcores
    b_per_w = BATCH // nw
    nchunk = b_per_w // CHUNK

    e_t = jnp.swapaxes(E_center, 0, 1)
    rpad = jnp.concatenate([R_center, R_offset], axis=1)

    def shard(x):
        return x.astype(jnp.int32).reshape(nw, nchunk, CHUNK)

    run = functools.partial(
        pl.kernel,
        out_type=jax.ShapeDtypeStruct((nw, nchunk, CHUNK), jnp.float32),
        mesh=plsc.VectorSubcoreMesh(core_axis_name="c", subcore_axis_name="s"),
        compiler_params=pltpu.CompilerParams(
            needs_layout_passes=False, disable_bounds_checks=True,
            use_tc_tiling_on_sc=False),
        scratch_types=[
            pltpu.VMEM((nchunk, CHUNK), jnp.int32),
            pltpu.VMEM((nchunk, CHUNK), jnp.int32),
            pltpu.VMEM((nchunk, CHUNK), jnp.int32),
            pltpu.VMEM((2, EMBED_DIM, CHUNK), jnp.float32),
            pltpu.VMEM((2, EMBED_DIM, CHUNK), jnp.float32),
            pltpu.VMEM((2, CHUNK, 2 * EMBED_DIM), jnp.float32),
            pltpu.VMEM((nchunk, CHUNK), jnp.float32),
            pltpu.SemaphoreType.DMA((2,)),
        ],
    )(_sc_body)

    out = run(e_t, rpad, shard(s), shard(r), shard(o))
    return out.reshape(BATCH)


# speedup vs baseline: 12.1810x; 12.1810x over previous
import functools

import jax
import jax.numpy as jnp
from jax import lax
from jax.experimental import pallas as pl
from jax.experimental.pallas import tpu as pltpu
from jax.experimental.pallas import tpu_sc as plsc

ALPHA = 0.2
BATCH = 16384
EMBED_DIM = 64
CHUNK = 128
NGROUP = CHUNK // 16
NENT = 1_000_000
TBLOCK = 2048
HALF = ((NENT // 2 + TBLOCK - 1) // TBLOCK) * TBLOCK


def _tc_pairs_body(x1_ref, x2_ref, y_ref):
    y_ref[:, :EMBED_DIM] = x1_ref[...].T
    y_ref[:, EMBED_DIM:] = x2_ref[...].T


def _pair_table(e_t):
    return pl.pallas_call(
        _tc_pairs_body,
        grid=(HALF // TBLOCK,),
        in_specs=[
            pl.BlockSpec((EMBED_DIM, TBLOCK), lambda i: (0, i)),
            pl.BlockSpec((EMBED_DIM, TBLOCK),
                         lambda i: (0, jnp.minimum(
                             i + HALF // TBLOCK,
                             (NENT + TBLOCK - 1) // TBLOCK - 1))),
        ],
        out_specs=pl.BlockSpec((TBLOCK, 2 * EMBED_DIM), lambda i: (i, 0)),
        out_shape=jax.ShapeDtypeStruct((HALF, 2 * EMBED_DIM), jnp.float32),
    )(e_t, e_t)


def _sc_body(e2_hbm, rpad_hbm, s_hbm, r_hbm, o_hbm, out_hbm,
             s_vm, o_vm, s_h, o_h, r_i, srow, orow, rrow, outbuf, sems):
    info = plsc.get_sparse_core_info()
    nw = info.num_cores * info.num_subcores
    b_per_w = BATCH // nw
    nchunk = b_per_w // CHUNK

    wid = lax.axis_index("s") * info.num_cores + lax.axis_index("c")
    lanes = lax.iota(jnp.int32, 16)

    pltpu.sync_copy(s_hbm.at[wid], s_vm)
    pltpu.sync_copy(o_hbm.at[wid], o_vm)
    pltpu.sync_copy(r_hbm.at[wid], r_i)

    for c in range(nchunk):
        for g in range(NGROUP):
            ksl = pl.ds(g * 16, 16)
            sv = s_vm[c, ksl]
            ov = o_vm[c, ksl]
            s_h[c, ksl] = sv - jnp.where(sv >= HALF, HALF, 0)
            o_h[c, ksl] = ov - jnp.where(ov >= HALF, HALF, 0)

    def issue(c, buf):
        pltpu.async_copy(rpad_hbm.at[r_i.at[c]], rrow.at[buf], sems.at[buf])
        pltpu.async_copy(e2_hbm.at[s_h.at[c]], srow.at[buf], sems.at[buf])
        pltpu.async_copy(e2_hbm.at[o_h.at[c]], orow.at[buf], sems.at[buf])

    def drain(buf):
        pltpu.make_async_copy(e2_hbm.at[pl.ds(0, CHUNK)], srow.at[buf],
                              sems.at[buf]).wait()
        pltpu.make_async_copy(e2_hbm.at[pl.ds(0, CHUNK)], orow.at[buf],
                              sems.at[buf]).wait()
        pltpu.make_async_copy(rpad_hbm.at[pl.ds(0, CHUNK)], rrow.at[buf],
                              sems.at[buf]).wait()

    def compute(c, buf):
        for bg in range(NGROUP):
            ksl = pl.ds(bg * 16, 16)
            brow = bg * 16 + lanes
            scol = jnp.where(s_vm[c, ksl] >= HALF, EMBED_DIM, 0)
            ocol = jnp.where(o_vm[c, ksl] >= HALF, EMBED_DIM, 0)

            def d_body(d, acc):
                dv = jnp.full((16,), 0, jnp.int32) + d
                sv = plsc.load_gather(srow.at[buf], [brow, scol + dv])
                ov = plsc.load_gather(orow.at[buf], [brow, ocol + dv])
                rc = plsc.load_gather(rrow.at[buf], [brow, dv])
                ro = plsc.load_gather(rrow.at[buf], [brow, dv + EMBED_DIM])
                t = ov - sv - rc
                off = jnp.maximum(ro, 0.0)
                a = jnp.abs(t)
                return acc + (jnp.maximum(a - off, 0.0)
                              + ALPHA * jnp.minimum(a, off))

            acc = lax.fori_loop(0, EMBED_DIM, d_body,
                                jnp.zeros((16,), jnp.float32))
            outbuf[c, ksl] = -acc

    issue(0, 0)

    def chunk_body(c, carry):
        buf = c % 2

        @pl.when(c + 1 < nchunk)
        def _():
            issue(c + 1, 1 - buf)

        drain(buf)
        compute(c, buf)
        return carry

    lax.fori_loop(0, nchunk, chunk_body, 0)
    pltpu.sync_copy(outbuf, out_hbm.at[wid])


def kernel(E_center, R_center, R_offset, s, r, o):
    info = plsc.get_sparse_core_info()
    nw = info.num_cores * info.num_subcores
    b_per_w = BATCH // nw
    nchunk = b_per_w // CHUNK

    e_t = jnp.swapaxes(E_center, 0, 1)
    e2 = _pair_table(e_t)
    rpad = jnp.concatenate([R_center, R_offset], axis=1)

    def shard(x):
        return x.astype(jnp.int32).reshape(nw, nchunk, CHUNK)

    run = functools.partial(
        pl.kernel,
        out_type=jax.ShapeDtypeStruct((nw, nchunk, CHUNK), jnp.float32),
        mesh=plsc.VectorSubcoreMesh(core_axis_name="c", subcore_axis_name="s"),
        compiler_params=pltpu.CompilerParams(
            needs_layout_passes=False, disable_bounds_checks=True),
        scratch_types=[
            pltpu.VMEM((nchunk, CHUNK), jnp.int32),
            pltpu.VMEM((nchunk, CHUNK), jnp.int32),
            pltpu.VMEM((nchunk, CHUNK), jnp.int32),
            pltpu.VMEM((nchunk, CHUNK), jnp.int32),
            pltpu.VMEM((nchunk, CHUNK), jnp.int32),
            pltpu.VMEM((2, CHUNK, 2 * EMBED_DIM), jnp.float32),
            pltpu.VMEM((2, CHUNK, 2 * EMBED_DIM), jnp.float32),
            pltpu.VMEM((2, CHUNK, 2 * EMBED_DIM), jnp.float32),
            pltpu.VMEM((nchunk, CHUNK), jnp.float32),
            pltpu.SemaphoreType.DMA((2,)),
        ],
    )(_sc_body)

    out = run(e2, rpad, shard(s), shard(r), shard(o))
    return out.reshape(BATCH)


# speedup vs baseline: 14.4850x; 1.1892x over previous
import functools

import jax
import jax.numpy as jnp
from jax import lax
from jax.experimental import pallas as pl
from jax.experimental.pallas import tpu as pltpu
from jax.experimental.pallas import tpu_sc as plsc

ALPHA = 0.2
BATCH = 16384
EMBED_DIM = 64
CHUNK = 128
NGROUP = CHUNK // 16
NENT = 1_000_000
TBLOCK = 4096
HALF = ((NENT // 2 + TBLOCK - 1) // TBLOCK) * TBLOCK


def _tc_pairs_body(x1_ref, x2_ref, y_ref):
    y_ref[:, :EMBED_DIM] = x1_ref[...].T
    y_ref[:, EMBED_DIM:] = x2_ref[...].T


def _pair_table(e_t):
    return pl.pallas_call(
        _tc_pairs_body,
        grid=(HALF // TBLOCK,),
        in_specs=[
            pl.BlockSpec((EMBED_DIM, TBLOCK), lambda i: (0, i)),
            pl.BlockSpec((EMBED_DIM, TBLOCK),
                         lambda i: (0, jnp.minimum(
                             i + HALF // TBLOCK,
                             (NENT + TBLOCK - 1) // TBLOCK - 1))),
        ],
        out_specs=pl.BlockSpec((TBLOCK, 2 * EMBED_DIM), lambda i: (i, 0)),
        out_shape=jax.ShapeDtypeStruct((HALF, 2 * EMBED_DIM), jnp.float32),
    )(e_t, e_t)


def _sc_body(e2_hbm, rpad_hbm, s_hbm, r_hbm, o_hbm, out_hbm,
             s_vm, o_vm, s_h, o_h, r_i, srow, orow, rrow, outbuf, sems):
    info = plsc.get_sparse_core_info()
    nw = info.num_cores * info.num_subcores
    b_per_w = BATCH // nw
    nchunk = b_per_w // CHUNK

    wid = lax.axis_index("s") * info.num_cores + lax.axis_index("c")
    lanes = lax.iota(jnp.int32, 16)

    pltpu.sync_copy(s_hbm.at[wid], s_vm)
    pltpu.sync_copy(o_hbm.at[wid], o_vm)
    pltpu.sync_copy(r_hbm.at[wid], r_i)

    for c in range(nchunk):
        for g in range(NGROUP):
            ksl = pl.ds(g * 16, 16)
            sv = s_vm[c, ksl]
            ov = o_vm[c, ksl]
            s_h[c, ksl] = sv - jnp.where(sv >= HALF, HALF, 0)
            o_h[c, ksl] = ov - jnp.where(ov >= HALF, HALF, 0)

    def issue(c, buf):
        pltpu.async_copy(rpad_hbm.at[r_i.at[c]], rrow.at[buf], sems.at[buf])
        pltpu.async_copy(e2_hbm.at[s_h.at[c]], srow.at[buf], sems.at[buf])
        pltpu.async_copy(e2_hbm.at[o_h.at[c]], orow.at[buf], sems.at[buf])

    def drain(buf):
        pltpu.make_async_copy(e2_hbm.at[pl.ds(0, CHUNK)], srow.at[buf],
                              sems.at[buf]).wait()
        pltpu.make_async_copy(e2_hbm.at[pl.ds(0, CHUNK)], orow.at[buf],
                              sems.at[buf]).wait()
        pltpu.make_async_copy(rpad_hbm.at[pl.ds(0, CHUNK)], rrow.at[buf],
                              sems.at[buf]).wait()

    def compute(c, buf):
        for bg in range(NGROUP):
            ksl = pl.ds(bg * 16, 16)
            brow = bg * 16 + lanes
            scol = jnp.where(s_vm[c, ksl] >= HALF, EMBED_DIM, 0)
            ocol = jnp.where(o_vm[c, ksl] >= HALF, EMBED_DIM, 0)

            def d_body(d, acc):
                dv = jnp.full((16,), 0, jnp.int32) + d
                sv = plsc.load_gather(srow.at[buf], [brow, scol + dv])
                ov = plsc.load_gather(orow.at[buf], [brow, ocol + dv])
                rc = plsc.load_gather(rrow.at[buf], [brow, dv])
                ro = plsc.load_gather(rrow.at[buf], [brow, dv + EMBED_DIM])
                t = ov - sv - rc
                off = jnp.maximum(ro, 0.0)
                a = jnp.abs(t)
                return acc + (jnp.maximum(a - off, 0.0)
                              + ALPHA * jnp.minimum(a, off))

            acc = lax.fori_loop(0, EMBED_DIM, d_body,
                                jnp.zeros((16,), jnp.float32))
            outbuf[c, ksl] = -acc

    issue(0, 0)

    def chunk_body(c, carry):
        buf = c % 2

        @pl.when(c + 1 < nchunk)
        def _():
            issue(c + 1, 1 - buf)

        drain(buf)
        compute(c, buf)
        return carry

    lax.fori_loop(0, nchunk, chunk_body, 0)
    pltpu.sync_copy(outbuf, out_hbm.at[wid])


def kernel(E_center, R_center, R_offset, s, r, o):
    info = plsc.get_sparse_core_info()
    nw = info.num_cores * info.num_subcores
    b_per_w = BATCH // nw
    nchunk = b_per_w // CHUNK

    e_t = jnp.swapaxes(E_center, 0, 1)
    e2 = _pair_table(e_t)
    rpad = jnp.concatenate([R_center, R_offset], axis=1)

    def shard(x):
        return x.astype(jnp.int32).reshape(nw, nchunk, CHUNK)

    run = functools.partial(
        pl.kernel,
        out_type=jax.ShapeDtypeStruct((nw, nchunk, CHUNK), jnp.float32),
        mesh=plsc.VectorSubcoreMesh(core_axis_name="c", subcore_axis_name="s"),
        compiler_params=pltpu.CompilerParams(
            needs_layout_passes=False, disable_bounds_checks=True),
        scratch_types=[
            pltpu.VMEM((nchunk, CHUNK), jnp.int32),
            pltpu.VMEM((nchunk, CHUNK), jnp.int32),
            pltpu.VMEM((nchunk, CHUNK), jnp.int32),
            pltpu.VMEM((nchunk, CHUNK), jnp.int32),
            pltpu.VMEM((nchunk, CHUNK), jnp.int32),
            pltpu.VMEM((2, CHUNK, 2 * EMBED_DIM), jnp.float32),
            pltpu.VMEM((2, CHUNK, 2 * EMBED_DIM), jnp.float32),
            pltpu.VMEM((2, CHUNK, 2 * EMBED_DIM), jnp.float32),
            pltpu.VMEM((nchunk, CHUNK), jnp.float32),
            pltpu.SemaphoreType.DMA((2,)),
        ],
    )(_sc_body)

    out = run(e2, rpad, shard(s), shard(r), shard(o))
    return out.reshape(BATCH)


# speedup vs baseline: 15.9626x; 1.1020x over previous
import functools

import jax
import jax.numpy as jnp
from jax import lax
from jax.experimental import pallas as pl
from jax.experimental.pallas import tpu as pltpu
from jax.experimental.pallas import tpu_sc as plsc

ALPHA = 0.2
BATCH = 16384
EMBED_DIM = 64
CHUNK = 128
NGROUP = CHUNK // 16
NENT = 1_000_000
TBLOCK = 8192
HALF = ((NENT // 2 + TBLOCK - 1) // TBLOCK) * TBLOCK


def _tc_pairs_body(x1_ref, x2_ref, y_ref):
    y_ref[:, :EMBED_DIM] = x1_ref[...].T
    y_ref[:, EMBED_DIM:] = x2_ref[...].T


def _pair_table(e_t):
    return pl.pallas_call(
        _tc_pairs_body,
        grid=(HALF // TBLOCK,),
        in_specs=[
            pl.BlockSpec((EMBED_DIM, TBLOCK), lambda i: (0, i)),
            pl.BlockSpec((EMBED_DIM, TBLOCK),
                         lambda i: (0, jnp.minimum(
                             i + HALF // TBLOCK,
                             (NENT + TBLOCK - 1) // TBLOCK - 1))),
        ],
        out_specs=pl.BlockSpec((TBLOCK, 2 * EMBED_DIM), lambda i: (i, 0)),
        out_shape=jax.ShapeDtypeStruct((HALF, 2 * EMBED_DIM), jnp.float32),
    )(e_t, e_t)


def _sc_body(e2_hbm, rpad_hbm, s_hbm, r_hbm, o_hbm, out_hbm,
             s_vm, o_vm, s_h, o_h, r_i, srow, orow, rrow, outbuf, sems):
    info = plsc.get_sparse_core_info()
    nw = info.num_cores * info.num_subcores
    b_per_w = BATCH // nw
    nchunk = b_per_w // CHUNK

    wid = lax.axis_index("s") * info.num_cores + lax.axis_index("c")
    lanes = lax.iota(jnp.int32, 16)

    pltpu.sync_copy(s_hbm.at[wid], s_vm)
    pltpu.sync_copy(o_hbm.at[wid], o_vm)
    pltpu.sync_copy(r_hbm.at[wid], r_i)

    for c in range(nchunk):
        for g in range(NGROUP):
            ksl = pl.ds(g * 16, 16)
            sv = s_vm[c, ksl]
            ov = o_vm[c, ksl]
            s_h[c, ksl] = sv - jnp.where(sv >= HALF, HALF, 0)
            o_h[c, ksl] = ov - jnp.where(ov >= HALF, HALF, 0)

    def issue(c, buf):
        pltpu.async_copy(rpad_hbm.at[r_i.at[c]], rrow.at[buf], sems.at[buf])
        pltpu.async_copy(e2_hbm.at[s_h.at[c]], srow.at[buf], sems.at[buf])
        pltpu.async_copy(e2_hbm.at[o_h.at[c]], orow.at[buf], sems.at[buf])

    def drain(buf):
        pltpu.make_async_copy(e2_hbm.at[pl.ds(0, CHUNK)], srow.at[buf],
                              sems.at[buf]).wait()
        pltpu.make_async_copy(e2_hbm.at[pl.ds(0, CHUNK)], orow.at[buf],
                              sems.at[buf]).wait()
        pltpu.make_async_copy(rpad_hbm.at[pl.ds(0, CHUNK)], rrow.at[buf],
                              sems.at[buf]).wait()

    def compute(c, buf):
        for bg in range(NGROUP):
            ksl = pl.ds(bg * 16, 16)
            brow = bg * 16 + lanes
            scol = jnp.where(s_vm[c, ksl] >= HALF, EMBED_DIM, 0)
            ocol = jnp.where(o_vm[c, ksl] >= HALF, EMBED_DIM, 0)

            def d_body(d, acc):
                dv = jnp.full((16,), 0, jnp.int32) + d
                sv = plsc.load_gather(srow.at[buf], [brow, scol + dv])
                ov = plsc.load_gather(orow.at[buf], [brow, ocol + dv])
                rc = plsc.load_gather(rrow.at[buf], [brow, dv])
                ro = plsc.load_gather(rrow.at[buf], [brow, dv + EMBED_DIM])
                t = ov - sv - rc
                off = jnp.maximum(ro, 0.0)
                a = jnp.abs(t)
                return acc + (jnp.maximum(a - off, 0.0)
                              + ALPHA * jnp.minimum(a, off))

            acc = lax.fori_loop(0, EMBED_DIM, d_body,
                                jnp.zeros((16,), jnp.float32))
            outbuf[c, ksl] = -acc

    issue(0, 0)

    def chunk_body(c, carry):
        buf = c % 2

        @pl.when(c + 1 < nchunk)
        def _():
            issue(c + 1, 1 - buf)

        drain(buf)
        compute(c, buf)
        return carry

    lax.fori_loop(0, nchunk, chunk_body, 0)
    pltpu.sync_copy(outbuf, out_hbm.at[wid])


def kernel(E_center, R_center, R_offset, s, r, o):
    info = plsc.get_sparse_core_info()
    nw = info.num_cores * info.num_subcores
    b_per_w = BATCH // nw
    nchunk = b_per_w // CHUNK

    e_t = jnp.swapaxes(E_center, 0, 1)
    e2 = _pair_table(e_t)
    rpad = jnp.concatenate([R_center, R_offset], axis=1)

    def shard(x):
        return x.astype(jnp.int32).reshape(nw, nchunk, CHUNK)

    run = functools.partial(
        pl.kernel,
        out_type=jax.ShapeDtypeStruct((nw, nchunk, CHUNK), jnp.float32),
        mesh=plsc.VectorSubcoreMesh(core_axis_name="c", subcore_axis_name="s"),
        compiler_params=pltpu.CompilerParams(
            needs_layout_passes=False, disable_bounds_checks=True),
        scratch_types=[
            pltpu.VMEM((nchunk, CHUNK), jnp.int32),
            pltpu.VMEM((nchunk, CHUNK), jnp.int32),
            pltpu.VMEM((nchunk, CHUNK), jnp.int32),
            pltpu.VMEM((nchunk, CHUNK), jnp.int32),
            pltpu.VMEM((nchunk, CHUNK), jnp.int32),
            pltpu.VMEM((2, CHUNK, 2 * EMBED_DIM), jnp.float32),
            pltpu.VMEM((2, CHUNK, 2 * EMBED_DIM), jnp.float32),
            pltpu.VMEM((2, CHUNK, 2 * EMBED_DIM), jnp.float32),
            pltpu.VMEM((nchunk, CHUNK), jnp.float32),
            pltpu.SemaphoreType.DMA((2,)),
        ],
    )(_sc_body)

    out = run(e2, rpad, shard(s), shard(r), shard(o))
    return out.reshape(BATCH)


# speedup vs baseline: 16.6786x; 1.0449x over previous
import functools

import jax
import jax.numpy as jnp
from jax import lax
from jax.experimental import pallas as pl
from jax.experimental.pallas import tpu as pltpu
from jax.experimental.pallas import tpu_sc as plsc

ALPHA = 0.2
BATCH = 16384
EMBED_DIM = 64
CHUNK = 128
NGROUP = CHUNK // 16
NENT = 1_000_000
TBLOCK = 16384
HALF = ((NENT // 2 + TBLOCK - 1) // TBLOCK) * TBLOCK


def _tc_pairs_body(x1_ref, x2_ref, y_ref):
    y_ref[:, :EMBED_DIM] = x1_ref[...].T
    y_ref[:, EMBED_DIM:] = x2_ref[...].T


def _pair_table(e_t):
    return pl.pallas_call(
        _tc_pairs_body,
        grid=(HALF // TBLOCK,),
        in_specs=[
            pl.BlockSpec((EMBED_DIM, TBLOCK), lambda i: (0, i)),
            pl.BlockSpec((EMBED_DIM, TBLOCK),
                         lambda i: (0, jnp.minimum(
                             i + HALF // TBLOCK,
                             (NENT + TBLOCK - 1) // TBLOCK - 1))),
        ],
        out_specs=pl.BlockSpec((TBLOCK, 2 * EMBED_DIM), lambda i: (i, 0)),
        out_shape=jax.ShapeDtypeStruct((HALF, 2 * EMBED_DIM), jnp.float32),
    )(e_t, e_t)


def _sc_body(e2_hbm, rpad_hbm, s_hbm, r_hbm, o_hbm, out_hbm,
             s_vm, o_vm, s_h, o_h, r_i, srow, orow, rrow, outbuf, sems):
    info = plsc.get_sparse_core_info()
    nw = info.num_cores * info.num_subcores
    b_per_w = BATCH // nw
    nchunk = b_per_w // CHUNK

    wid = lax.axis_index("s") * info.num_cores + lax.axis_index("c")
    lanes = lax.iota(jnp.int32, 16)

    pltpu.sync_copy(s_hbm.at[wid], s_vm)
    pltpu.sync_copy(o_hbm.at[wid], o_vm)
    pltpu.sync_copy(r_hbm.at[wid], r_i)

    for c in range(nchunk):
        for g in range(NGROUP):
            ksl = pl.ds(g * 16, 16)
            sv = s_vm[c, ksl]
            ov = o_vm[c, ksl]
            s_h[c, ksl] = sv - jnp.where(sv >= HALF, HALF, 0)
            o_h[c, ksl] = ov - jnp.where(ov >= HALF, HALF, 0)

    def issue(c, buf):
        pltpu.async_copy(rpad_hbm.at[r_i.at[c]], rrow.at[buf], sems.at[buf])
        pltpu.async_copy(e2_hbm.at[s_h.at[c]], srow.at[buf], sems.at[buf])
        pltpu.async_copy(e2_hbm.at[o_h.at[c]], orow.at[buf], sems.at[buf])

    def drain(buf):
        pltpu.make_async_copy(e2_hbm.at[pl.ds(0, CHUNK)], srow.at[buf],
                              sems.at[buf]).wait()
        pltpu.make_async_copy(e2_hbm.at[pl.ds(0, CHUNK)], orow.at[buf],
                              sems.at[buf]).wait()
        pltpu.make_async_copy(rpad_hbm.at[pl.ds(0, CHUNK)], rrow.at[buf],
                              sems.at[buf]).wait()

    def compute(c, buf):
        for bg in range(NGROUP):
            ksl = pl.ds(bg * 16, 16)
            brow = bg * 16 + lanes
            scol = jnp.where(s_vm[c, ksl] >= HALF, EMBED_DIM, 0)
            ocol = jnp.where(o_vm[c, ksl] >= HALF, EMBED_DIM, 0)

            def d_body(d, acc):
                dv = jnp.full((16,), 0, jnp.int32) + d
                sv = plsc.load_gather(srow.at[buf], [brow, scol + dv])
                ov = plsc.load_gather(orow.at[buf], [brow, ocol + dv])
                rc = plsc.load_gather(rrow.at[buf], [brow, dv])
                ro = plsc.load_gather(rrow.at[buf], [brow, dv + EMBED_DIM])
                t = ov - sv - rc
                off = jnp.maximum(ro, 0.0)
                a = jnp.abs(t)
                return acc + (jnp.maximum(a - off, 0.0)
                              + ALPHA * jnp.minimum(a, off))

            acc = lax.fori_loop(0, EMBED_DIM, d_body,
                                jnp.zeros((16,), jnp.float32))
            outbuf[c, ksl] = -acc

    issue(0, 0)

    def chunk_body(c, carry):
        buf = c % 2

        @pl.when(c + 1 < nchunk)
        def _():
            issue(c + 1, 1 - buf)

        drain(buf)
        compute(c, buf)
        return carry

    lax.fori_loop(0, nchunk, chunk_body, 0)
    pltpu.sync_copy(outbuf, out_hbm.at[wid])


def kernel(E_center, R_center, R_offset, s, r, o):
    info = plsc.get_sparse_core_info()
    nw = info.num_cores * info.num_subcores
    b_per_w = BATCH // nw
    nchunk = b_per_w // CHUNK

    e_t = jnp.swapaxes(E_center, 0, 1)
    e2 = _pair_table(e_t)
    rpad = jnp.concatenate([R_center, R_offset], axis=1)

    def shard(x):
        return x.astype(jnp.int32).reshape(nw, nchunk, CHUNK)

    run = functools.partial(
        pl.kernel,
        out_type=jax.ShapeDtypeStruct((nw, nchunk, CHUNK), jnp.float32),
        mesh=plsc.VectorSubcoreMesh(core_axis_name="c", subcore_axis_name="s"),
        compiler_params=pltpu.CompilerParams(
            needs_layout_passes=False, disable_bounds_checks=True),
        scratch_types=[
            pltpu.VMEM((nchunk, CHUNK), jnp.int32),
            pltpu.VMEM((nchunk, CHUNK), jnp.int32),
            pltpu.VMEM((nchunk, CHUNK), jnp.int32),
            pltpu.VMEM((nchunk, CHUNK), jnp.int32),
            pltpu.VMEM((nchunk, CHUNK), jnp.int32),
            pltpu.VMEM((2, CHUNK, 2 * EMBED_DIM), jnp.float32),
            pltpu.VMEM((2, CHUNK, 2 * EMBED_DIM), jnp.float32),
            pltpu.VMEM((2, CHUNK, 2 * EMBED_DIM), jnp.float32),
            pltpu.VMEM((nchunk, CHUNK), jnp.float32),
            pltpu.SemaphoreType.DMA((2,)),
        ],
    )(_sc_body)

    out = run(e2, rpad, shard(s), shard(r), shard(o))
    return out.reshape(BATCH)
